# Initial kernel scaffold; baseline (speedup 1.0000x reference)
import jax
import jax.numpy as jnp
from jax.experimental import pallas as pl


def kernel(boxes, labels, masks):
    raise NotImplementedError("write your pallas kernel here")


# speedup vs baseline: 21.6276x; 21.6276x over previous
import jax
import jax.numpy as jnp
from jax import lax
from jax.experimental import pallas as pl

_SCALE_RANGES = ((1, 96), (48, 192), (96, 384), (192, 768), (384, 2048))
_FPN_SIZE = (40, 36, 24, 16, 12)
_SMAX = 40
_IMG = 512
_SIG = 0.1
_NOBJ = 32


def _stats_body(mask_ref, stats_ref):
    m = mask_ref[0, 0]
    ind = jnp.where(m != 0.0, jnp.float32(1.0), jnp.float32(0.0))
    si = lax.broadcasted_iota(jnp.int32, (8, _IMG), 0)
    ki = lax.broadcasted_iota(jnp.int32, (8, _IMG), 1).astype(jnp.float32)
    w = jnp.where(si == 0, jnp.float32(1.0),
                  jnp.where(si == 1, ki, jnp.float32(0.0)))
    r = lax.dot_general(w, ind, (((1,), (0,)), ((), ())),
                        preferred_element_type=jnp.float32,
                        precision=lax.Precision.HIGHEST)
    per_col = r[0:1, :].astype(jnp.int32)
    wrow = r[1:2, :].astype(jnp.int32)
    ci = lax.broadcasted_iota(jnp.int32, (1, _IMG), 1)
    count = jnp.sum(per_col)
    colsum = jnp.sum(per_col * ci)
    rowsum = jnp.sum(wrow)
    lane = lax.broadcasted_iota(jnp.int32, (1, 128), 1)
    vec = jnp.where(lane == 0, count,
                    jnp.where(lane == 1, rowsum,
                              jnp.where(lane == 2, colsum, 0)))
    stats_ref[0, 0] = vec


def _assign_body(stats_ref, boxes_ref, labels_ref, cat_ref, pt_ref):
    st = stats_ref[0]
    cnt = st[0:1, :]
    rsum = st[1:2, :]
    csum = st[2:3, :]
    b = boxes_ref[0]
    x1, y1, x2, y2 = b[0:1, :], b[1:2, :], b[2:3, :], b[3:4, :]
    labs = labels_ref[0]
    one = jnp.float32(1.0)
    hl = y2 - y1 + one
    wl = x2 - x1 + one
    area = jnp.sqrt(hl * wl)
    safe = jnp.maximum(cnt.astype(jnp.float32), one)
    has = cnt > 0
    half = jnp.float32(0.5)
    y_mean = jnp.where(has, rsum.astype(jnp.float32) / safe, half * (y1 + y2))
    x_mean = jnp.where(has, csum.astype(jnp.float32) / safe, half * (x1 + x2))
    sig = jnp.float32(_SIG)
    lim = jnp.float32(_IMG - 1)
    zero = jnp.float32(0.0)
    left = jnp.clip(x_mean - sig * wl, zero, lim)
    right = jnp.clip(x_mean + sig * wl, zero, lim)
    top = jnp.clip(y_mean - sig * hl, zero, lim)
    bot = jnp.clip(y_mean + sig * hl, zero, lim)
    inf = jnp.float32(jnp.inf)
    for i, s in enumerate(_FPN_SIZE):
        scale = jnp.float32(_IMG / s)
        lo, hi = _SCALE_RANGES[i]
        in_r = (area >= jnp.float32(lo)) & (area <= jnp.float32(hi))
        smax = jnp.float32(s - 1)
        p_l = jnp.clip(jnp.floor(left / scale), zero, smax).astype(jnp.int32)
        p_r = jnp.clip(jnp.floor(right / scale), zero, smax).astype(jnp.int32)
        p_t = jnp.clip(jnp.floor(top / scale), zero, smax).astype(jnp.int32)
        p_b = jnp.clip(jnp.floor(bot / scale), zero, smax).astype(jnp.int32)
        keyv = jnp.where(in_r, area, inf)
        rr = lax.broadcasted_iota(jnp.int32, (_SMAX, _SMAX), 0)
        cc = lax.broadcasted_iota(jnp.int32, (_SMAX, _SMAX), 1)
        best = jnp.full((_SMAX, _SMAX), inf, jnp.float32)
        bpt = jnp.full((_SMAX, _SMAX), -1, jnp.int32)
        bcat = jnp.zeros((_SMAX, _SMAX), jnp.int32)
        for o in range(_NOBJ):
            kko = keyv[0:1, o:o + 1]
            rect = ((rr >= p_t[0:1, o:o + 1]) & (rr <= p_b[0:1, o:o + 1])
                    & (cc >= p_l[0:1, o:o + 1]) & (cc <= p_r[0:1, o:o + 1]))
            upd = rect & (kko <= best) & (kko < inf)
            best = jnp.where(upd, kko, best)
            bpt = jnp.where(upd, o, bpt)
            bcat = jnp.where(upd, labs[0:1, o:o + 1], bcat)
        cat_ref[0, i] = bcat
        pt_ref[0, i] = bpt


def kernel(boxes, labels, masks):
    n, obj = masks.shape[0], masks.shape[1]
    boxes = jnp.asarray(boxes, dtype=jnp.float32)
    labels = jnp.asarray(labels, dtype=jnp.int32)
    masks = jnp.asarray(masks, dtype=jnp.float32)
    stats = pl.pallas_call(
        _stats_body,
        grid=(n, obj),
        in_specs=[pl.BlockSpec((1, 1, _IMG, _IMG), lambda b, o: (b, o, 0, 0))],
        out_specs=pl.BlockSpec((1, 1, 1, 128), lambda b, o: (b, o, 0, 0)),
        out_shape=jax.ShapeDtypeStruct((n, obj, 1, 128), jnp.int32),
    )(masks)
    stats_t = jnp.pad(stats[:, :, 0, :3].transpose(0, 2, 1),
                      ((0, 0), (0, 5), (0, 128 - obj)))
    boxes_p = jnp.pad(boxes.transpose(0, 2, 1),
                      ((0, 0), (0, 4), (0, 128 - obj)))
    labels_p = jnp.pad(labels.reshape(n, 1, obj),
                       ((0, 0), (0, 0), (0, 128 - obj)))
    nl = len(_FPN_SIZE)
    catp, ptp = pl.pallas_call(
        _assign_body,
        grid=(n,),
        in_specs=[
            pl.BlockSpec((1, 8, 128), lambda b: (b, 0, 0)),
            pl.BlockSpec((1, 8, 128), lambda b: (b, 0, 0)),
            pl.BlockSpec((1, 1, 128), lambda b: (b, 0, 0)),
        ],
        out_specs=[
            pl.BlockSpec((1, nl, _SMAX, _SMAX), lambda b: (b, 0, 0, 0)),
            pl.BlockSpec((1, nl, _SMAX, _SMAX), lambda b: (b, 0, 0, 0)),
        ],
        out_shape=[
            jax.ShapeDtypeStruct((n, nl, _SMAX, _SMAX), jnp.int32),
            jax.ShapeDtypeStruct((n, nl, _SMAX, _SMAX), jnp.int32),
        ],
    )(stats_t, boxes_p, labels_p)
    cats = [catp[:, i, :s, :s].reshape(n, s * s) for i, s in enumerate(_FPN_SIZE)]
    pts = [ptp[:, i, :s, :s].reshape(n, s * s) for i, s in enumerate(_FPN_SIZE)]
    return jnp.concatenate(cats, axis=1), jnp.concatenate(pts, axis=1)


# speedup vs baseline: 26.1353x; 1.2084x over previous
import jax
import jax.numpy as jnp
from jax import lax
from jax.experimental import pallas as pl

_SCALE_RANGES = ((1, 96), (48, 192), (96, 384), (192, 768), (384, 2048))
_FPN_SIZE = (40, 36, 24, 16, 12)
_SMAX = 40
_IMG = 512
_SIG = 0.1
_NOBJ = 32


def _stats_body(mask_ref, stats_ref):
    m = mask_ref[0, 0]
    ind = jnp.where(m != 0.0, jnp.float32(1.0), jnp.float32(0.0))
    si = lax.broadcasted_iota(jnp.int32, (8, _IMG), 0)
    ki = lax.broadcasted_iota(jnp.int32, (8, _IMG), 1)
    w = jnp.where(si == 0, 1,
                  jnp.where(si == 1, ki & ~1,
                            jnp.where(si == 2, ki & 1, 0))).astype(jnp.float32)
    r = lax.dot_general(w, ind, (((1,), (0,)), ((), ())),
                        preferred_element_type=jnp.float32)
    per_col = r[0:1, :].astype(jnp.int32)
    ci = lax.broadcasted_iota(jnp.int32, (1, _IMG), 1)
    count = jnp.sum(per_col)
    colsum = jnp.sum(per_col * ci)
    rowsum = jnp.sum(r[1:2, :].astype(jnp.int32)) + jnp.sum(r[2:3, :].astype(jnp.int32))
    o = pl.program_id(1)
    lane = lax.broadcasted_iota(jnp.int32, (8, 128), 1)
    row = lax.broadcasted_iota(jnp.int32, (8, 128), 0)
    hit = lane == o
    prev = stats_ref[0]
    stats_ref[0] = jnp.where(hit & (row == 0), count,
                             jnp.where(hit & (row == 1), rowsum,
                                       jnp.where(hit & (row == 2), colsum, prev)))


def _assign_body(stats_ref, boxes_ref, labels_ref, cat_ref, pt_ref):
    st = stats_ref[0]
    cnt = st[0:1, :]
    rsum = st[1:2, :]
    csum = st[2:3, :]
    b = boxes_ref[0]
    x1, y1, x2, y2 = b[0:1, :], b[1:2, :], b[2:3, :], b[3:4, :]
    labs = labels_ref[0]
    one = jnp.float32(1.0)
    hl = y2 - y1 + one
    wl = x2 - x1 + one
    area = jnp.sqrt(hl * wl)
    safe = jnp.maximum(cnt.astype(jnp.float32), one)
    has = cnt > 0
    half = jnp.float32(0.5)
    y_mean = jnp.where(has, rsum.astype(jnp.float32) / safe, half * (y1 + y2))
    x_mean = jnp.where(has, csum.astype(jnp.float32) / safe, half * (x1 + x2))
    sig = jnp.float32(_SIG)
    lim = jnp.float32(_IMG - 1)
    zero = jnp.float32(0.0)
    left = jnp.clip(x_mean - sig * wl, zero, lim)
    right = jnp.clip(x_mean + sig * wl, zero, lim)
    top = jnp.clip(y_mean - sig * hl, zero, lim)
    bot = jnp.clip(y_mean + sig * hl, zero, lim)
    inf = jnp.float32(jnp.inf)
    for i, s in enumerate(_FPN_SIZE):
        scale = jnp.float32(_IMG / s)
        lo, hi = _SCALE_RANGES[i]
        in_r = (area >= jnp.float32(lo)) & (area <= jnp.float32(hi))
        smax = jnp.float32(s - 1)
        p_l = jnp.clip(jnp.floor(left / scale), zero, smax).astype(jnp.int32)
        p_r = jnp.clip(jnp.floor(right / scale), zero, smax).astype(jnp.int32)
        p_t = jnp.clip(jnp.floor(top / scale), zero, smax).astype(jnp.int32)
        p_b = jnp.clip(jnp.floor(bot / scale), zero, smax).astype(jnp.int32)
        keyv = jnp.where(in_r, area, inf)
        rr = lax.broadcasted_iota(jnp.int32, (_SMAX, _SMAX), 0)
        cc = lax.broadcasted_iota(jnp.int32, (_SMAX, _SMAX), 1)
        best = jnp.full((_SMAX, _SMAX), inf, jnp.float32)
        bpt = jnp.full((_SMAX, _SMAX), -1, jnp.int32)
        bcat = jnp.zeros((_SMAX, _SMAX), jnp.int32)
        for o in range(_NOBJ):
            kko = keyv[0:1, o:o + 1]
            rect = ((rr >= p_t[0:1, o:o + 1]) & (rr <= p_b[0:1, o:o + 1])
                    & (cc >= p_l[0:1, o:o + 1]) & (cc <= p_r[0:1, o:o + 1]))
            upd = rect & (kko <= best) & (kko < inf)
            best = jnp.where(upd, kko, best)
            bpt = jnp.where(upd, o, bpt)
            bcat = jnp.where(upd, labs[0:1, o:o + 1], bcat)
        cat_ref[0, i] = bcat
        pt_ref[0, i] = bpt


def kernel(boxes, labels, masks):
    n, obj = masks.shape[0], masks.shape[1]
    boxes = jnp.asarray(boxes, dtype=jnp.float32)
    labels = jnp.asarray(labels, dtype=jnp.int32)
    masks = jnp.asarray(masks, dtype=jnp.float32)
    stats_t = pl.pallas_call(
        _stats_body,
        grid=(n, obj),
        in_specs=[pl.BlockSpec((1, 1, _IMG, _IMG), lambda b, o: (b, o, 0, 0))],
        out_specs=pl.BlockSpec((1, 8, 128), lambda b, o: (b, 0, 0)),
        out_shape=jax.ShapeDtypeStruct((n, 8, 128), jnp.int32),
    )(masks)
    boxes_p = jnp.pad(boxes.transpose(0, 2, 1),
                      ((0, 0), (0, 4), (0, 128 - obj)))
    labels_p = jnp.pad(labels.reshape(n, 1, obj),
                       ((0, 0), (0, 0), (0, 128 - obj)))
    nl = len(_FPN_SIZE)
    catp, ptp = pl.pallas_call(
        _assign_body,
        grid=(n,),
        in_specs=[
            pl.BlockSpec((1, 8, 128), lambda b: (b, 0, 0)),
            pl.BlockSpec((1, 8, 128), lambda b: (b, 0, 0)),
            pl.BlockSpec((1, 1, 128), lambda b: (b, 0, 0)),
        ],
        out_specs=[
            pl.BlockSpec((1, nl, _SMAX, _SMAX), lambda b: (b, 0, 0, 0)),
            pl.BlockSpec((1, nl, _SMAX, _SMAX), lambda b: (b, 0, 0, 0)),
        ],
        out_shape=[
            jax.ShapeDtypeStruct((n, nl, _SMAX, _SMAX), jnp.int32),
            jax.ShapeDtypeStruct((n, nl, _SMAX, _SMAX), jnp.int32),
        ],
    )(stats_t, boxes_p, labels_p)
    cats = [catp[:, i, :s, :s].reshape(n, s * s) for i, s in enumerate(_FPN_SIZE)]
    pts = [ptp[:, i, :s, :s].reshape(n, s * s) for i, s in enumerate(_FPN_SIZE)]
    return jnp.concatenate(cats, axis=1), jnp.concatenate(pts, axis=1)


# speedup vs baseline: 47.8734x; 1.8318x over previous
import jax
import jax.numpy as jnp
from jax import lax
from jax.experimental import pallas as pl

_SCALE_RANGES = ((1, 96), (48, 192), (96, 384), (192, 768), (384, 2048))
_FPN_SIZE = (40, 36, 24, 16, 12)
_SMAX = 40
_IMG = 512
_SIG = 0.1
_NOBJ = 32


_MPB = 4


def _stats_body(mask_ref, stats_ref):
    si = lax.broadcasted_iota(jnp.int32, (8, _IMG), 0)
    ki = lax.broadcasted_iota(jnp.int32, (8, _IMG), 1)
    w = jnp.where(si == 0, 1,
                  jnp.where(si == 1, ki & ~1,
                            jnp.where(si == 2, ki & 1, 0))).astype(jnp.float32)
    ci = lax.broadcasted_iota(jnp.int32, (1, _IMG), 1)
    lane = lax.broadcasted_iota(jnp.int32, (8, 128), 1)
    row = lax.broadcasted_iota(jnp.int32, (8, 128), 0)
    o_base = pl.program_id(1) * _MPB
    acc = stats_ref[0]
    for j in range(_MPB):
        m = mask_ref[0, j]
        ind = jnp.where(m != 0.0, jnp.float32(1.0), jnp.float32(0.0))
        r = lax.dot_general(w, ind, (((1,), (0,)), ((), ())),
                            preferred_element_type=jnp.float32)
        per_col = r[0:1, :].astype(jnp.int32)
        count = jnp.sum(per_col)
        colsum = jnp.sum(per_col * ci)
        rowsum = (jnp.sum(r[1:2, :].astype(jnp.int32))
                  + jnp.sum(r[2:3, :].astype(jnp.int32)))
        hit = lane == o_base + j
        acc = jnp.where(hit & (row == 0), count,
                        jnp.where(hit & (row == 1), rowsum,
                                  jnp.where(hit & (row == 2), colsum, acc)))
    stats_ref[0] = acc


def _assign_body(stats_ref, boxes_ref, labels_ref, cat_ref, pt_ref):
    st = stats_ref[0]
    cnt = st[0:1, :]
    rsum = st[1:2, :]
    csum = st[2:3, :]
    b = boxes_ref[0]
    x1, y1, x2, y2 = b[0:1, :], b[1:2, :], b[2:3, :], b[3:4, :]
    labs = labels_ref[0]
    one = jnp.float32(1.0)
    hl = y2 - y1 + one
    wl = x2 - x1 + one
    area = jnp.sqrt(hl * wl)
    safe = jnp.maximum(cnt.astype(jnp.float32), one)
    has = cnt > 0
    half = jnp.float32(0.5)
    y_mean = jnp.where(has, rsum.astype(jnp.float32) / safe, half * (y1 + y2))
    x_mean = jnp.where(has, csum.astype(jnp.float32) / safe, half * (x1 + x2))
    sig = jnp.float32(_SIG)
    lim = jnp.float32(_IMG - 1)
    zero = jnp.float32(0.0)
    left = jnp.clip(x_mean - sig * wl, zero, lim)
    right = jnp.clip(x_mean + sig * wl, zero, lim)
    top = jnp.clip(y_mean - sig * hl, zero, lim)
    bot = jnp.clip(y_mean + sig * hl, zero, lim)
    inf = jnp.float32(jnp.inf)
    for i, s in enumerate(_FPN_SIZE):
        scale = jnp.float32(_IMG / s)
        lo, hi = _SCALE_RANGES[i]
        in_r = (area >= jnp.float32(lo)) & (area <= jnp.float32(hi))
        smax = jnp.float32(s - 1)
        p_l = jnp.clip(jnp.floor(left / scale), zero, smax).astype(jnp.int32)
        p_r = jnp.clip(jnp.floor(right / scale), zero, smax).astype(jnp.int32)
        p_t = jnp.clip(jnp.floor(top / scale), zero, smax).astype(jnp.int32)
        p_b = jnp.clip(jnp.floor(bot / scale), zero, smax).astype(jnp.int32)
        keyv = jnp.where(in_r, area, inf)
        rr = lax.broadcasted_iota(jnp.int32, (_SMAX, _SMAX), 0)
        cc = lax.broadcasted_iota(jnp.int32, (_SMAX, _SMAX), 1)
        best = jnp.full((_SMAX, _SMAX), inf, jnp.float32)
        bpt = jnp.full((_SMAX, _SMAX), -1, jnp.int32)
        bcat = jnp.zeros((_SMAX, _SMAX), jnp.int32)
        for o in range(_NOBJ):
            kko = keyv[0:1, o:o + 1]
            rect = ((rr >= p_t[0:1, o:o + 1]) & (rr <= p_b[0:1, o:o + 1])
                    & (cc >= p_l[0:1, o:o + 1]) & (cc <= p_r[0:1, o:o + 1]))
            upd = rect & (kko <= best) & (kko < inf)
            best = jnp.where(upd, kko, best)
            bpt = jnp.where(upd, o, bpt)
            bcat = jnp.where(upd, labs[0:1, o:o + 1], bcat)
        cat_ref[0, i] = bcat
        pt_ref[0, i] = bpt


def kernel(boxes, labels, masks):
    n, obj = masks.shape[0], masks.shape[1]
    boxes = jnp.asarray(boxes, dtype=jnp.float32)
    labels = jnp.asarray(labels, dtype=jnp.int32)
    masks = jnp.asarray(masks, dtype=jnp.float32)
    stats_t = pl.pallas_call(
        _stats_body,
        grid=(n, obj // _MPB),
        in_specs=[pl.BlockSpec((1, _MPB, _IMG, _IMG), lambda b, o: (b, o, 0, 0))],
        out_specs=pl.BlockSpec((1, 8, 128), lambda b, o: (b, 0, 0)),
        out_shape=jax.ShapeDtypeStruct((n, 8, 128), jnp.int32),
    )(masks)
    boxes_p = jnp.pad(boxes.transpose(0, 2, 1),
                      ((0, 0), (0, 4), (0, 128 - obj)))
    labels_p = jnp.pad(labels.reshape(n, 1, obj),
                       ((0, 0), (0, 0), (0, 128 - obj)))
    nl = len(_FPN_SIZE)
    catp, ptp = pl.pallas_call(
        _assign_body,
        grid=(n,),
        in_specs=[
            pl.BlockSpec((1, 8, 128), lambda b: (b, 0, 0)),
            pl.BlockSpec((1, 8, 128), lambda b: (b, 0, 0)),
            pl.BlockSpec((1, 1, 128), lambda b: (b, 0, 0)),
        ],
        out_specs=[
            pl.BlockSpec((1, nl, _SMAX, _SMAX), lambda b: (b, 0, 0, 0)),
            pl.BlockSpec((1, nl, _SMAX, _SMAX), lambda b: (b, 0, 0, 0)),
        ],
        out_shape=[
            jax.ShapeDtypeStruct((n, nl, _SMAX, _SMAX), jnp.int32),
            jax.ShapeDtypeStruct((n, nl, _SMAX, _SMAX), jnp.int32),
        ],
    )(stats_t, boxes_p, labels_p)
    cats = [catp[:, i, :s, :s].reshape(n, s * s) for i, s in enumerate(_FPN_SIZE)]
    pts = [ptp[:, i, :s, :s].reshape(n, s * s) for i, s in enumerate(_FPN_SIZE)]
    return jnp.concatenate(cats, axis=1), jnp.concatenate(pts, axis=1)


# speedup vs baseline: 52.8924x; 1.1048x over previous
import jax
import jax.numpy as jnp
from jax import lax
from jax.experimental import pallas as pl

_SCALE_RANGES = ((1, 96), (48, 192), (96, 384), (192, 768), (384, 2048))
_FPN_SIZE = (40, 36, 24, 16, 12)
_SMAX = 40
_IMG = 512
_SIG = 0.1
_NOBJ = 32


_MPB = 8


def _stats_body(mask_ref, stats_ref):
    si = lax.broadcasted_iota(jnp.int32, (8, _IMG), 0)
    ki = lax.broadcasted_iota(jnp.int32, (8, _IMG), 1)
    w = jnp.where(si == 0, 1,
                  jnp.where(si == 1, ki & ~1,
                            jnp.where(si == 2, ki & 1, 0))).astype(jnp.float32)
    ci = lax.broadcasted_iota(jnp.int32, (1, _IMG), 1)
    lane = lax.broadcasted_iota(jnp.int32, (8, 128), 1)
    row = lax.broadcasted_iota(jnp.int32, (8, 128), 0)
    o_base = pl.program_id(1) * _MPB
    acc = stats_ref[0]
    for j in range(_MPB):
        m = mask_ref[0, j]
        ind = jnp.where(m != 0.0, jnp.float32(1.0), jnp.float32(0.0))
        r = lax.dot_general(w, ind, (((1,), (0,)), ((), ())),
                            preferred_element_type=jnp.float32)
        per_col = r[0:1, :].astype(jnp.int32)
        count = jnp.sum(per_col)
        colsum = jnp.sum(per_col * ci)
        rowsum = (jnp.sum(r[1:2, :].astype(jnp.int32))
                  + jnp.sum(r[2:3, :].astype(jnp.int32)))
        hit = lane == o_base + j
        acc = jnp.where(hit & (row == 0), count,
                        jnp.where(hit & (row == 1), rowsum,
                                  jnp.where(hit & (row == 2), colsum, acc)))
    stats_ref[0] = acc


def _assign_body(stats_ref, boxes_ref, labels_ref, cat_ref, pt_ref):
    st = stats_ref[0]
    cnt = st[0:1, :]
    rsum = st[1:2, :]
    csum = st[2:3, :]
    b = boxes_ref[0]
    x1, y1, x2, y2 = b[0:1, :], b[1:2, :], b[2:3, :], b[3:4, :]
    labs = labels_ref[0]
    one = jnp.float32(1.0)
    hl = y2 - y1 + one
    wl = x2 - x1 + one
    area = jnp.sqrt(hl * wl)
    safe = jnp.maximum(cnt.astype(jnp.float32), one)
    has = cnt > 0
    half = jnp.float32(0.5)
    y_mean = jnp.where(has, rsum.astype(jnp.float32) / safe, half * (y1 + y2))
    x_mean = jnp.where(has, csum.astype(jnp.float32) / safe, half * (x1 + x2))
    sig = jnp.float32(_SIG)
    lim = jnp.float32(_IMG - 1)
    zero = jnp.float32(0.0)
    left = jnp.clip(x_mean - sig * wl, zero, lim)
    right = jnp.clip(x_mean + sig * wl, zero, lim)
    top = jnp.clip(y_mean - sig * hl, zero, lim)
    bot = jnp.clip(y_mean + sig * hl, zero, lim)
    inf = jnp.float32(jnp.inf)
    for i, s in enumerate(_FPN_SIZE):
        scale = jnp.float32(_IMG / s)
        lo, hi = _SCALE_RANGES[i]
        in_r = (area >= jnp.float32(lo)) & (area <= jnp.float32(hi))
        smax = jnp.float32(s - 1)
        p_l = jnp.clip(jnp.floor(left / scale), zero, smax).astype(jnp.int32)
        p_r = jnp.clip(jnp.floor(right / scale), zero, smax).astype(jnp.int32)
        p_t = jnp.clip(jnp.floor(top / scale), zero, smax).astype(jnp.int32)
        p_b = jnp.clip(jnp.floor(bot / scale), zero, smax).astype(jnp.int32)
        keyv = jnp.where(in_r, area, inf)
        rr = lax.broadcasted_iota(jnp.int32, (_SMAX, _SMAX), 0)
        cc = lax.broadcasted_iota(jnp.int32, (_SMAX, _SMAX), 1)
        best = jnp.full((_SMAX, _SMAX), inf, jnp.float32)
        bpt = jnp.full((_SMAX, _SMAX), -1, jnp.int32)
        bcat = jnp.zeros((_SMAX, _SMAX), jnp.int32)
        for o in range(_NOBJ):
            kko = keyv[0:1, o:o + 1]
            rect = ((rr >= p_t[0:1, o:o + 1]) & (rr <= p_b[0:1, o:o + 1])
                    & (cc >= p_l[0:1, o:o + 1]) & (cc <= p_r[0:1, o:o + 1]))
            upd = rect & (kko <= best) & (kko < inf)
            best = jnp.where(upd, kko, best)
            bpt = jnp.where(upd, o, bpt)
            bcat = jnp.where(upd, labs[0:1, o:o + 1], bcat)
        cat_ref[0, i] = bcat
        pt_ref[0, i] = bpt


def kernel(boxes, labels, masks):
    n, obj = masks.shape[0], masks.shape[1]
    boxes = jnp.asarray(boxes, dtype=jnp.float32)
    labels = jnp.asarray(labels, dtype=jnp.int32)
    masks = jnp.asarray(masks, dtype=jnp.float32)
    stats_t = pl.pallas_call(
        _stats_body,
        grid=(n, obj // _MPB),
        in_specs=[pl.BlockSpec((1, _MPB, _IMG, _IMG), lambda b, o: (b, o, 0, 0))],
        out_specs=pl.BlockSpec((1, 8, 128), lambda b, o: (b, 0, 0)),
        out_shape=jax.ShapeDtypeStruct((n, 8, 128), jnp.int32),
    )(masks)
    boxes_p = jnp.pad(boxes.transpose(0, 2, 1),
                      ((0, 0), (0, 4), (0, 128 - obj)))
    labels_p = jnp.pad(labels.reshape(n, 1, obj),
                       ((0, 0), (0, 0), (0, 128 - obj)))
    nl = len(_FPN_SIZE)
    catp, ptp = pl.pallas_call(
        _assign_body,
        grid=(n,),
        in_specs=[
            pl.BlockSpec((1, 8, 128), lambda b: (b, 0, 0)),
            pl.BlockSpec((1, 8, 128), lambda b: (b, 0, 0)),
            pl.BlockSpec((1, 1, 128), lambda b: (b, 0, 0)),
        ],
        out_specs=[
            pl.BlockSpec((1, nl, _SMAX, _SMAX), lambda b: (b, 0, 0, 0)),
            pl.BlockSpec((1, nl, _SMAX, _SMAX), lambda b: (b, 0, 0, 0)),
        ],
        out_shape=[
            jax.ShapeDtypeStruct((n, nl, _SMAX, _SMAX), jnp.int32),
            jax.ShapeDtypeStruct((n, nl, _SMAX, _SMAX), jnp.int32),
        ],
    )(stats_t, boxes_p, labels_p)
    cats = [catp[:, i, :s, :s].reshape(n, s * s) for i, s in enumerate(_FPN_SIZE)]
    pts = [ptp[:, i, :s, :s].reshape(n, s * s) for i, s in enumerate(_FPN_SIZE)]
    return jnp.concatenate(cats, axis=1), jnp.concatenate(pts, axis=1)
